# baseline (speedup 1.0000x reference)
import jax
import jax.numpy as jnp
from jax.experimental import pallas as pl
from jax.experimental.pallas import tpu as pltpu


def _body(x_ref, w1_ref, b1_ref, w2_ref, b2_ref, e2_ref, e2t_ref,
          w3_ref, b3_ref, w4_ref, b4_ref,
          recon_ref, ze_ref, embout_ref, *, K, P):
    f32 = jnp.float32
    x = x_ref[...]
    h1 = jnp.maximum(
        jnp.dot(x, w1_ref[...], preferred_element_type=f32) + b1_ref[...], 0.0)
    ze = jnp.dot(h1, w2_ref[...], preferred_element_type=f32) + b2_ref[...]
    ze_ref[...] = ze

    e2mat = e2_ref[...]
    e2c = jnp.sum(e2mat * e2mat, axis=0, keepdims=True)
    scores = e2c - 2.0 * jnp.dot(ze, e2mat, preferred_element_type=f32)

    iota = jax.lax.broadcasted_iota(jnp.int32, (scores.shape[0], K), 1)
    ohs = []
    for p in range(P):
        s = scores[:, p * K:(p + 1) * K]
        m = jnp.min(s, axis=1, keepdims=True)
        cand = jnp.where(s == m, iota, K)
        kmin = jnp.min(cand, axis=1, keepdims=True)
        ohs.append((iota == kmin).astype(f32))
    oh = jnp.concatenate(ohs, axis=1)
    zq = jnp.dot(oh, e2t_ref[...], preferred_element_type=f32)
    embout_ref[...] = zq

    h3 = jnp.maximum(
        jnp.dot(zq, w3_ref[...], preferred_element_type=f32) + b3_ref[...], 0.0)
    logits = jnp.dot(h3, w4_ref[...], preferred_element_type=f32) + b4_ref[...]
    recon_ref[...] = jax.nn.sigmoid(logits)


def kernel(x, W1, b1, W2, b2, W3, b3, W4, b4, emb_weight):
    B, L = x.shape
    D, K = emb_weight.shape
    H = W2.shape[0]
    P = H // D
    F1 = W1.shape[0]
    BB = 256

    W1T, W2T, W3T, W4T = W1.T, W2.T, W3.T, W4.T
    b1r, b2r, b3r, b4r = (b.reshape(1, -1) for b in (b1, b2, b3, b4))
    E2 = jnp.zeros((H, P * K), x.dtype)
    for p in range(P):
        E2 = E2.at[p::P, p * K:(p + 1) * K].set(emb_weight)
    E2T = E2.T

    import functools
    grid = (B // BB,)
    full = lambda shape: pl.BlockSpec(shape, lambda i: (0, 0))
    row = lambda shape: pl.BlockSpec(shape, lambda i: (i, 0))

    recon, ze, embout = pl.pallas_call(
        functools.partial(_body, K=K, P=P),
        grid=grid,
        in_specs=[
            row((BB, L)),
            full((L, F1)), full((1, F1)),
            full((F1, H)), full((1, H)),
            full((H, P * K)), full((P * K, H)),
            full((H, F1)), full((1, F1)),
            full((F1, L)), full((1, L)),
        ],
        out_specs=(row((BB, L)), row((BB, H)), row((BB, H))),
        out_shape=(
            jax.ShapeDtypeStruct((B, L), x.dtype),
            jax.ShapeDtypeStruct((B, H), x.dtype),
            jax.ShapeDtypeStruct((B, H), x.dtype),
        ),
        compiler_params=pltpu.CompilerParams(
            dimension_semantics=("arbitrary",)),
    )(x, W1T, b1r, W2T, b2r, E2, E2T, W3T, b3r, W4T, b4r)

    return recon, ze.reshape(B, D, P), embout


# speedup vs baseline: 1.0172x; 1.0172x over previous
import jax
import jax.numpy as jnp
from jax.experimental import pallas as pl
from jax.experimental.pallas import tpu as pltpu


def _body(x_ref, w1_ref, b1_ref, w2_ref, b2_ref, e2_ref, e2t_ref,
          w3_ref, b3_ref, w4_ref, b4_ref,
          recon_ref, ze_ref, embout_ref, *, K, P):
    f32 = jnp.float32
    x = x_ref[...]
    h1 = jnp.maximum(
        jnp.dot(x, w1_ref[...], preferred_element_type=f32) + b1_ref[...], 0.0)
    ze = jnp.dot(h1, w2_ref[...], preferred_element_type=f32) + b2_ref[...]
    ze_ref[...] = ze

    e2mat = e2_ref[...]
    e2c = jnp.sum(e2mat * e2mat, axis=0, keepdims=True)
    scores = e2c - 2.0 * jnp.dot(ze, e2mat, preferred_element_type=f32)

    iota = jax.lax.broadcasted_iota(jnp.int32, (scores.shape[0], K), 1)
    ohs = []
    for p in range(P):
        s = scores[:, p * K:(p + 1) * K]
        m = jnp.min(s, axis=1, keepdims=True)
        cand = jnp.where(s == m, iota, K)
        kmin = jnp.min(cand, axis=1, keepdims=True)
        ohs.append((iota == kmin).astype(f32))
    oh = jnp.concatenate(ohs, axis=1)
    zq = jnp.dot(oh, e2t_ref[...], preferred_element_type=f32)
    embout_ref[...] = zq

    h3 = jnp.maximum(
        jnp.dot(zq, w3_ref[...], preferred_element_type=f32) + b3_ref[...], 0.0)
    logits = jnp.dot(h3, w4_ref[...], preferred_element_type=f32) + b4_ref[...]
    recon_ref[...] = jax.nn.sigmoid(logits)


def kernel(x, W1, b1, W2, b2, W3, b3, W4, b4, emb_weight):
    B, L = x.shape
    D, K = emb_weight.shape
    H = W2.shape[0]
    P = H // D
    F1 = W1.shape[0]
    BB = 512

    W1T, W2T, W3T, W4T = W1.T, W2.T, W3.T, W4.T
    b1r, b2r, b3r, b4r = (b.reshape(1, -1) for b in (b1, b2, b3, b4))
    E2 = jnp.zeros((H, P * K), x.dtype)
    for p in range(P):
        E2 = E2.at[p::P, p * K:(p + 1) * K].set(emb_weight)
    E2T = E2.T

    import functools
    grid = (B // BB,)
    full = lambda shape: pl.BlockSpec(shape, lambda i: (0, 0))
    row = lambda shape: pl.BlockSpec(shape, lambda i: (i, 0))

    recon, ze, embout = pl.pallas_call(
        functools.partial(_body, K=K, P=P),
        grid=grid,
        in_specs=[
            row((BB, L)),
            full((L, F1)), full((1, F1)),
            full((F1, H)), full((1, H)),
            full((H, P * K)), full((P * K, H)),
            full((H, F1)), full((1, F1)),
            full((F1, L)), full((1, L)),
        ],
        out_specs=(row((BB, L)), row((BB, H)), row((BB, H))),
        out_shape=(
            jax.ShapeDtypeStruct((B, L), x.dtype),
            jax.ShapeDtypeStruct((B, H), x.dtype),
            jax.ShapeDtypeStruct((B, H), x.dtype),
        ),
        compiler_params=pltpu.CompilerParams(
            dimension_semantics=("arbitrary",)),
    )(x, W1T, b1r, W2T, b2r, E2, E2T, W3T, b3r, W4T, b4r)

    return recon, ze.reshape(B, D, P), embout
